# baseline (speedup 1.0000x reference)
import functools

import jax
import jax.numpy as jnp
from jax import lax
from jax.experimental import pallas as pl
from jax.experimental.pallas import tpu as pltpu
from jax.experimental.pallas import tpu_sc as plsc

N = 10000
E = 320000
D = 128

NC = 2
NS = 16
NW = NC * NS

C = 80
EW = E // NW
NCH = EW // C

RPB = C
NB = N // RPB
KMAX = -(-NB // NS)


@functools.partial(
    pl.kernel,
    out_type=jax.ShapeDtypeStruct((NC, N, D), jnp.float32),
    mesh=plsc.VectorSubcoreMesh(
        core_axis_name="c", subcore_axis_name="s",
        num_cores=NC, num_subcores=NS),
    scratch_types=[
        pltpu.VMEM((EW,), jnp.int32),
        pltpu.VMEM((EW,), jnp.int32),
        pltpu.VMEM((C, D), jnp.float32),
        pltpu.VMEM((C, D), jnp.float32),
        pltpu.VMEM((C, D), jnp.float32),
        pltpu.VMEM_SHARED((N, D), jnp.float32),
        pltpu.SemaphoreType.DMA,
        pltpu.SemaphoreType.DMA,
        pltpu.SemaphoreType.DMA,
        pltpu.SemaphoreType.DMA,
    ],
)
def _sc_aggregate(x_hbm, src_hbm, dst_hbm, out_hbm,
                  sall, dall, rows, rows_b, rows_c, agg,
                  sem, sem_b, sem_c, sem_k):
    cid = lax.axis_index("c")
    sid = lax.axis_index("s")
    wid = cid * NS + sid

    pltpu.sync_copy(src_hbm.at[pl.ds(wid * EW, EW)], sall)
    pltpu.sync_copy(dst_hbm.at[pl.ds(wid * EW, EW)], dall)

    def zrow(i, _):
        def zlane(j, _):
            rows[i, pl.ds(j * 16, 16)] = jnp.zeros((16,), jnp.float32)
            return 0
        return lax.fori_loop(0, D // 16, zlane, 0)
    lax.fori_loop(0, C, zrow, 0)

    def zcp(k, _):
        cb = sid + k * NS
        @pl.when(cb < NB)
        def _():
            pltpu.async_copy(rows, agg.at[pl.ds(cb * RPB, RPB)], sem_k)
        return 0
    lax.fori_loop(0, KMAX, zcp, 0)

    def zdr(k, _):
        cb = sid + k * NS
        @pl.when(cb < NB)
        def _():
            pltpu.make_async_copy(
                rows, agg.at[pl.ds(cb * RPB, RPB)], sem_k).wait()
        return 0
    lax.fori_loop(0, KMAX, zdr, 0)
    plsc.subcore_barrier()

    pltpu.async_copy(x_hbm.at[sall.at[pl.ds(0 * C, C)]], rows, sem)
    pltpu.async_copy(x_hbm.at[sall.at[pl.ds(1 * C, C)]], rows_b, sem_b)
    pltpu.async_copy(x_hbm.at[sall.at[pl.ds(2 * C, C)]], rows_c, sem_c)

    def chunk_triple(i, _):
        j = 3 * i
        for (buf, sm, o) in ((rows, sem, 0), (rows_b, sem_b, 1),
                             (rows_c, sem_c, 2)):
            pltpu.make_async_copy(
                x_hbm.at[sall.at[pl.ds((j + o) * C, C)]], buf, sm).wait()
            pltpu.sync_copy(
                buf, agg.at[dall.at[pl.ds((j + o) * C, C)]], add=True)

            @pl.when(j + o + 3 < NCH)
            def _():
                pltpu.async_copy(
                    x_hbm.at[sall.at[pl.ds((j + o + 3) * C, C)]], buf, sm)
        return 0
    lax.fori_loop(0, NCH // 3, chunk_triple, 0)

    for (buf, sm, jj) in ((rows, sem, NCH - 2), (rows_b, sem_b, NCH - 1)):
        pltpu.make_async_copy(
            x_hbm.at[sall.at[pl.ds(jj * C, C)]], buf, sm).wait()
        pltpu.sync_copy(buf, agg.at[dall.at[pl.ds(jj * C, C)]], add=True)
    plsc.subcore_barrier()

    def ocp(k, _):
        cb = sid + k * NS
        @pl.when(cb < NB)
        def _():
            r0 = cb * RPB
            pltpu.async_copy(agg.at[pl.ds(r0, RPB)],
                             out_hbm.at[cid, pl.ds(r0, RPB)], sem_k)
        return 0
    lax.fori_loop(0, KMAX, ocp, 0)

    def odr(k, _):
        cb = sid + k * NS
        @pl.when(cb < NB)
        def _():
            r0 = cb * RPB
            pltpu.make_async_copy(agg.at[pl.ds(r0, RPB)],
                                  out_hbm.at[cid, pl.ds(r0, RPB)],
                                  sem_k).wait()
        return 0
    lax.fori_loop(0, KMAX, odr, 0)


BR = 2000


def _tc_update_body(x_ref, p_ref, w_ref, b_ref, o_ref):
    h = x_ref[...] + p_ref[0] + p_ref[1]
    acc = jnp.dot(h, w_ref[...], preferred_element_type=jnp.float32)
    o_ref[...] = jnp.maximum(acc + b_ref[...], 0.0)


def _tc_update(x, parts, W, b2):
    return pl.pallas_call(
        _tc_update_body,
        grid=(N // BR,),
        in_specs=[
            pl.BlockSpec((BR, D), lambda i: (i, 0)),
            pl.BlockSpec((NC, BR, D), lambda i: (0, i, 0)),
            pl.BlockSpec((D, D), lambda i: (0, 0)),
            pl.BlockSpec((1, D), lambda i: (0, 0)),
        ],
        out_specs=pl.BlockSpec((BR, D), lambda i: (i, 0)),
        out_shape=jax.ShapeDtypeStruct((N, D), jnp.float32),
    )(x, parts, W, b2)


def kernel(x, edge_index, W, b):
    ei = edge_index.astype(jnp.int32)
    parts = _sc_aggregate(x, ei[0], ei[1])
    return _tc_update(x, parts, W, b.reshape(1, D))
